# baseline (speedup 1.0000x reference)
import functools

import jax
import jax.numpy as jnp
from jax import lax
from jax.experimental import pallas as pl
from jax.experimental.pallas import tpu as pltpu
from jax.experimental.pallas import tpu_sc as plsc


def _routed_slice(rows_total, chans, width, routes, num_batches):
    info = plsc.get_sparse_core_info()
    nc, ns, lanes = info.num_cores, info.num_subcores, info.num_lanes
    nw = nc * ns
    assert rows_total % nw == 0
    rows_per_w = rows_total // nw
    rows_per_batch = rows_total // num_batches
    assert rows_per_batch % rows_per_w == 0
    w_per_batch = rows_per_batch // rows_per_w
    win = 2 * 128

    mesh = plsc.VectorSubcoreMesh(core_axis_name="c", subcore_axis_name="s")

    @functools.partial(
        pl.kernel,
        mesh=mesh,
        compiler_params=pltpu.CompilerParams(needs_layout_passes=False,
                                             use_tc_tiling_on_sc=True),
        out_type=jax.ShapeDtypeStruct((rows_total, width), jnp.float32),
        scratch_types=[
            pltpu.VMEM((routes,), jnp.float32),
            pltpu.VMEM((4, 64, win), jnp.float32),
            pltpu.VMEM((2, 64, width), jnp.float32),
            pltpu.SemaphoreType.DMA,
            pltpu.SemaphoreType.DMA,
        ],
    )
    def k(table_hbm, routing_hbm, out_hbm, routing_v, win_v, comp_v, sem_g,
          sem_w):
        wid = lax.axis_index("s") * nc + lax.axis_index("c")
        base_row = wid * rows_per_w
        b = wid // w_per_batch

        pltpu.sync_copy(routing_hbm.at[b], routing_v)
        l_ids = lax.iota(jnp.int32, lanes)
        v = plsc.load_gather(routing_v, [l_ids & (routes - 1)])
        mx = plsc.load_gather(routing_v, [jnp.zeros((lanes,), jnp.int32)])
        for r in range(1, routes):
            mx = jnp.maximum(
                mx, plsc.load_gather(routing_v, [jnp.full((lanes,), r, jnp.int32)]))
        first = plsc.all_reduce_ffs((v == mx) & (l_ids < routes))
        route = jnp.int32(0)
        for r in range(1, routes):
            route = lax.select(jnp.any(first == r), jnp.int32(r), route)

        c_win = (route + route // 2) * 128
        off = route * width - c_win
        n_ch = rows_per_w // 64
        gathers = [
            pltpu.async_copy(
                table_hbm.at[pl.ds(base_row + i * 64, 64),
                             pl.ds(c_win, win)],
                win_v.at[i], sem_g)
            for i in range(n_ch)
        ]
        src_cols = [l_ids + (off + cix * lanes) for cix in range(width // lanes)]
        dst_cols = [l_ids + (cix * lanes) for cix in range(width // lanes)]

        writes = []
        for i in range(n_ch):
            gathers[i].wait()
            if i >= 2:
                writes[i - 2].wait()
            win_i = win_v.at[i]
            comp_i = comp_v.at[i % 2]

            @plsc.parallel_loop(0, 64, unroll=4)
            def body(r, win_i=win_i, comp_i=comp_i):
                r_ids = jnp.full((lanes,), r, jnp.int32)
                for cix in range(width // lanes):
                    vals = plsc.load_gather(win_i, [r_ids, src_cols[cix]])
                    plsc.store_scatter(comp_i, [r_ids, dst_cols[cix]], vals)
            writes.append(
                pltpu.async_copy(comp_i,
                                 out_hbm.at[pl.ds(base_row + i * 64, 64)],
                                 sem_w))
        for wr in writes[-2:]:
            wr.wait()

    return k


def kernel(inputs, routing_x):
    bsz, h, w_sp, c = inputs.shape
    routes = routing_x.shape[-1]
    width = c // routes
    rows_total = bsz * h * w_sp
    table = inputs.reshape(rows_total, c)
    out = _routed_slice(rows_total, c, width, routes, bsz)(table, routing_x)
    return out.reshape(bsz, h, w_sp, width)


# speedup vs baseline: 1.0265x; 1.0265x over previous
import functools

import jax
import jax.numpy as jnp
from jax import lax
from jax.experimental import pallas as pl
from jax.experimental.pallas import tpu as pltpu
from jax.experimental.pallas import tpu_sc as plsc


def _routed_slice(rows_total, chans, width, routes, num_batches):
    info = plsc.get_sparse_core_info()
    nc, ns, lanes = info.num_cores, info.num_subcores, info.num_lanes
    nw = nc * ns
    assert rows_total % nw == 0
    rows_per_w = rows_total // nw
    rows_per_batch = rows_total // num_batches
    assert rows_per_batch % rows_per_w == 0
    w_per_batch = rows_per_batch // rows_per_w
    win = 2 * 128

    mesh = plsc.VectorSubcoreMesh(core_axis_name="c", subcore_axis_name="s")

    @functools.partial(
        pl.kernel,
        mesh=mesh,
        compiler_params=pltpu.CompilerParams(needs_layout_passes=False,
                                             use_tc_tiling_on_sc=True),
        out_type=jax.ShapeDtypeStruct((rows_total, width), jnp.float32),
        scratch_types=[
            pltpu.VMEM((routes,), jnp.float32),
            pltpu.VMEM((4, 64, win), jnp.float32),
            pltpu.VMEM((2, 64, width), jnp.float32),
            pltpu.SemaphoreType.DMA,
            pltpu.SemaphoreType.DMA,
        ],
    )
    def k(table_hbm, routing_hbm, out_hbm, routing_v, win_v, comp_v, sem_g,
          sem_w):
        wid = lax.axis_index("s") * nc + lax.axis_index("c")
        base_row = wid * rows_per_w
        b = wid // w_per_batch

        pltpu.sync_copy(routing_hbm.at[b], routing_v)
        l_ids = lax.iota(jnp.int32, lanes)
        v = plsc.load_gather(routing_v, [l_ids & (routes - 1)])
        mx = plsc.load_gather(routing_v, [jnp.zeros((lanes,), jnp.int32)])
        for r in range(1, routes):
            mx = jnp.maximum(
                mx, plsc.load_gather(routing_v, [jnp.full((lanes,), r, jnp.int32)]))
        first = plsc.all_reduce_ffs((v == mx) & (l_ids < routes))
        route = jnp.int32(0)
        for r in range(1, routes):
            route = lax.select(jnp.any(first == r), jnp.int32(r), route)

        c_win = (route + route // 2) * 128
        off = route * width - c_win
        n_ch = rows_per_w // 64
        gathers = [
            pltpu.async_copy(
                table_hbm.at[pl.ds(base_row + i * 64, 64),
                             pl.ds(c_win, win)],
                win_v.at[i], sem_g)
            for i in range(n_ch)
        ]
        src_cols = [l_ids + (off + cix * lanes) for cix in range(width // lanes)]
        dst_cols = [l_ids + (cix * lanes) for cix in range(width // lanes)]

        writes = []
        for i in range(n_ch):
            gathers[i].wait()
            if i >= 2:
                writes[i - 2].wait()
            win_i = win_v.at[i]
            comp_i = comp_v.at[i % 2]

            @plsc.parallel_loop(0, 64, unroll=2)
            def body(r, win_i=win_i, comp_i=comp_i):
                r_ids = jnp.full((lanes,), r, jnp.int32)
                for cix in range(width // lanes):
                    vals = plsc.load_gather(win_i, [r_ids, src_cols[cix]])
                    plsc.store_scatter(comp_i, [r_ids, dst_cols[cix]], vals)
            writes.append(
                pltpu.async_copy(comp_i,
                                 out_hbm.at[pl.ds(base_row + i * 64, 64)],
                                 sem_w))
        for wr in writes[-2:]:
            wr.wait()

    return k


def kernel(inputs, routing_x):
    bsz, h, w_sp, c = inputs.shape
    routes = routing_x.shape[-1]
    width = c // routes
    rows_total = bsz * h * w_sp
    table = inputs.reshape(rows_total, c)
    out = _routed_slice(rows_total, c, width, routes, bsz)(table, routing_x)
    return out.reshape(bsz, h, w_sp, width)


# speedup vs baseline: 1.0512x; 1.0240x over previous
import functools

import jax
import jax.numpy as jnp
from jax import lax
from jax.experimental import pallas as pl
from jax.experimental.pallas import tpu as pltpu
from jax.experimental.pallas import tpu_sc as plsc


def _routed_slice(rows_total, chans, width, routes, num_batches):
    info = plsc.get_sparse_core_info()
    nc, ns, lanes = info.num_cores, info.num_subcores, info.num_lanes
    nw = nc * ns
    assert rows_total % nw == 0
    rows_per_w = rows_total // nw
    rows_per_batch = rows_total // num_batches
    assert rows_per_batch % rows_per_w == 0
    w_per_batch = rows_per_batch // rows_per_w
    win = 2 * 128

    mesh = plsc.VectorSubcoreMesh(core_axis_name="c", subcore_axis_name="s")

    @functools.partial(
        pl.kernel,
        mesh=mesh,
        compiler_params=pltpu.CompilerParams(needs_layout_passes=False,
                                             use_tc_tiling_on_sc=True),
        out_type=jax.ShapeDtypeStruct((rows_total, width), jnp.float32),
        scratch_types=[
            pltpu.VMEM((routes,), jnp.float32),
            pltpu.VMEM((4, 64, win), jnp.float32),
            pltpu.VMEM((2, 64, width), jnp.float32),
            pltpu.SemaphoreType.DMA,
            pltpu.SemaphoreType.DMA,
        ],
    )
    def k(table_hbm, routing_hbm, out_hbm, routing_v, win_v, comp_v, sem_g,
          sem_w):
        wid = lax.axis_index("s") * nc + lax.axis_index("c")
        base_row = wid * rows_per_w
        b = wid // w_per_batch

        pltpu.sync_copy(routing_hbm.at[b], routing_v)
        l_ids = lax.iota(jnp.int32, lanes)
        v = plsc.load_gather(routing_v, [l_ids & (routes - 1)])
        mx = plsc.load_gather(routing_v, [jnp.zeros((lanes,), jnp.int32)])
        for r in range(1, routes):
            mx = jnp.maximum(
                mx, plsc.load_gather(routing_v, [jnp.full((lanes,), r, jnp.int32)]))
        first = plsc.all_reduce_ffs((v == mx) & (l_ids < routes))
        route = jnp.int32(0)
        for r in range(1, routes):
            route = lax.select(jnp.any(first == r), jnp.int32(r), route)

        c_win = (route + route // 2) * 128
        off = route * width - c_win
        n_ch = rows_per_w // 64
        gathers = [
            pltpu.async_copy(
                table_hbm.at[pl.ds(base_row + i * 64, 64),
                             pl.ds(c_win, win)],
                win_v.at[i], sem_g)
            for i in range(n_ch)
        ]
        src_cols = [l_ids + (off + cix * lanes) for cix in range(width // lanes)]
        dst_cols = [l_ids + (cix * lanes) for cix in range(width // lanes)]

        writes = []
        for i in range(n_ch):
            gathers[i].wait()
            if i >= 2:
                writes[i - 2].wait()
            win_i = win_v.at[i]
            comp_i = comp_v.at[i % 2]

            @plsc.parallel_loop(0, 64, unroll=1)
            def body(r, win_i=win_i, comp_i=comp_i):
                r_ids = jnp.full((lanes,), r, jnp.int32)
                for cix in range(width // lanes):
                    vals = plsc.load_gather(win_i, [r_ids, src_cols[cix]])
                    plsc.store_scatter(comp_i, [r_ids, dst_cols[cix]], vals)
            writes.append(
                pltpu.async_copy(comp_i,
                                 out_hbm.at[pl.ds(base_row + i * 64, 64)],
                                 sem_w))
        for wr in writes[-2:]:
            wr.wait()

    return k


def kernel(inputs, routing_x):
    bsz, h, w_sp, c = inputs.shape
    routes = routing_x.shape[-1]
    width = c // routes
    rows_total = bsz * h * w_sp
    table = inputs.reshape(rows_total, c)
    out = _routed_slice(rows_total, c, width, routes, bsz)(table, routing_x)
    return out.reshape(bsz, h, w_sp, width)
